# baseline (speedup 1.0000x reference)
import functools

import jax
import jax.numpy as jnp
from jax import lax
from jax.experimental import pallas as pl
from jax.experimental.pallas import tpu as pltpu
from jax.experimental.pallas import tpu_sc as plsc

NUM_E = 26
VOC = 1000
D_E = 128
D_O = 512
B_SZ = 4096
L_SZ = 20
TOKENS = B_SZ * L_SZ
NC = 2
NS = 16
NW = NC * NS
TPW = TOKENS // NW
CHUNK = 4
NCHUNK = TPW // CHUNK


def _build_fused_table(tables, w):
    w3 = w.reshape(NUM_E, D_E, D_O)

    def body(t_ref, w_ref, f_ref):
        f_ref[...] = jnp.dot(
            t_ref[0], w_ref[0], preferred_element_type=jnp.float32
        )[None]

    f = pl.pallas_call(
        body,
        grid=(NUM_E,),
        in_specs=[
            pl.BlockSpec((1, VOC, D_E), lambda i: (i, 0, 0)),
            pl.BlockSpec((1, D_E, D_O), lambda i: (i, 0, 0)),
        ],
        out_specs=pl.BlockSpec((1, VOC, D_O), lambda i: (i, 0, 0)),
        out_shape=jax.ShapeDtypeStruct((NUM_E, VOC, D_O), jnp.float32),
    )(tables, w3)
    return f.reshape(NUM_E * VOC, D_O)


@functools.partial(
    pl.kernel,
    mesh=plsc.VectorSubcoreMesh(core_axis_name="c", subcore_axis_name="s"),
    out_type=jax.ShapeDtypeStruct((TOKENS, D_O), jnp.float32),
    scratch_types=[
        pltpu.VMEM((CHUNK * NUM_E,), jnp.int32),
        pltpu.VMEM((CHUNK * NUM_E, D_O), jnp.float32),
        pltpu.VMEM((CHUNK, D_O), jnp.float32),
        pltpu.VMEM((D_O,), jnp.float32),
        pltpu.SemaphoreType.DMA,
    ],
)
def _sc_lookup(rows_hbm, bias_hbm, f_hbm, out_hbm, idx_v, rows_v, out_v,
               bias_v, sem):
    cid = lax.axis_index("c")
    sid = lax.axis_index("s")
    wid = sid * NC + cid
    tok0 = wid * TPW
    pltpu.sync_copy(bias_hbm, bias_v)

    def chunk_body(k, carry):
        t0 = tok0 + k * CHUNK
        pltpu.sync_copy(rows_hbm.at[pl.ds(t0 * NUM_E, CHUNK * NUM_E)], idx_v)
        pltpu.async_copy(f_hbm.at[idx_v], rows_v, sem).wait()
        for t in range(CHUNK):
            base = t * NUM_E
            for v in range(D_O // 16):
                sl = pl.ds(v * 16, 16)
                acc = rows_v[base, sl] + bias_v[sl]
                for j in range(1, NUM_E):
                    acc = acc + rows_v[base + j, sl]
                out_v[t, sl] = acc
        pltpu.sync_copy(out_v, out_hbm.at[pl.ds(t0, CHUNK)])
        return carry

    lax.fori_loop(0, NCHUNK, chunk_body, 0)


def kernel(inputs, tables, W, b):
    f = _build_fused_table(tables, W)
    idx = inputs.reshape(TOKENS, NUM_E)
    rows = (idx + jnp.arange(NUM_E, dtype=jnp.int32) * VOC).reshape(-1)
    out = _sc_lookup(rows, b, f)
    return out.reshape(B_SZ, L_SZ, D_O)


# speedup vs baseline: 3.2331x; 3.2331x over previous
import functools

import jax
import jax.numpy as jnp
from jax import lax
from jax.experimental import pallas as pl
from jax.experimental.pallas import tpu as pltpu
from jax.experimental.pallas import tpu_sc as plsc

NUM_E = 26
VOC = 1000
D_E = 128
D_O = 512
B_SZ = 4096
L_SZ = 20
TOKENS = B_SZ * L_SZ
NC = 2
NS = 16
NW = NC * NS
R_TOT = NUM_E * TOKENS
RPW = R_TOT // NW
C2 = 128
NCH2 = RPW // C2
TB = 2048


@functools.partial(
    pl.kernel,
    mesh=plsc.VectorSubcoreMesh(core_axis_name="c", subcore_axis_name="s"),
    out_type=jax.ShapeDtypeStruct((R_TOT, D_E), jnp.float32),
    scratch_types=[
        pltpu.VMEM((C2,), jnp.int32),
        pltpu.VMEM((C2, D_E), jnp.float32),
        pltpu.SemaphoreType.DMA,
    ],
)
def _sc_gather(rows_hbm, tab_hbm, x_hbm, idx_v, buf_v, sem):
    cid = lax.axis_index("c")
    sid = lax.axis_index("s")
    wid = sid * NC + cid
    r0 = wid * RPW

    def chunk_body(k, carry):
        off = r0 + k * C2
        pltpu.sync_copy(rows_hbm.at[pl.ds(off, C2)], idx_v)
        pltpu.async_copy(tab_hbm.at[idx_v], buf_v, sem).wait()
        pltpu.sync_copy(buf_v, x_hbm.at[pl.ds(off, C2)])
        return carry

    lax.fori_loop(0, NCH2, chunk_body, 0)


def _tc_matmul(x3, w3, bias2):
    def body(x_ref, w_ref, b_ref, o_ref):
        i = pl.program_id(1)
        part = jnp.dot(x_ref[0].astype(jnp.bfloat16), w_ref[0],
                       preferred_element_type=jnp.float32)

        @pl.when(i == 0)
        def _():
            o_ref[...] = part + b_ref[...]

        @pl.when(i > 0)
        def _():
            o_ref[...] += part

    return pl.pallas_call(
        body,
        grid=(TOKENS // TB, NUM_E),
        in_specs=[
            pl.BlockSpec((1, TB, D_E), lambda t, i: (i, t, 0)),
            pl.BlockSpec((1, D_E, D_O), lambda t, i: (i, 0, 0)),
            pl.BlockSpec((1, D_O), lambda t, i: (0, 0)),
        ],
        out_specs=pl.BlockSpec((TB, D_O), lambda t, i: (t, 0)),
        out_shape=jax.ShapeDtypeStruct((TOKENS, D_O), jnp.float32),
    )(x3, w3, bias2)


def kernel(inputs, tables, W, b):
    tab2 = tables.reshape(NUM_E * VOC, D_E)
    idx_t = inputs.reshape(TOKENS, NUM_E).T
    rows = (idx_t + jnp.arange(NUM_E, dtype=jnp.int32)[:, None] * VOC).reshape(-1)
    x = _sc_gather(rows, tab2)
    x3 = x.reshape(NUM_E, TOKENS, D_E)
    w3 = W.astype(jnp.bfloat16).reshape(NUM_E, D_E, D_O)
    out = _tc_matmul(x3, w3, b.reshape(1, D_O))
    return out.reshape(B_SZ, L_SZ, D_O)


# speedup vs baseline: 4.5340x; 1.4024x over previous
import functools

import jax
import jax.numpy as jnp
from jax import lax
from jax.experimental import pallas as pl
from jax.experimental.pallas import tpu as pltpu
from jax.experimental.pallas import tpu_sc as plsc

NUM_E = 26
VOC = 1000
D_E = 128
D_O = 512
B_SZ = 4096
L_SZ = 20
TOKENS = B_SZ * L_SZ
NC = 2
NS = 16
NW = NC * NS
R_TOT = NUM_E * TOKENS
RPW = R_TOT // NW
C2 = 128
NCH2 = RPW // C2
TB = 512


@functools.partial(
    pl.kernel,
    mesh=plsc.VectorSubcoreMesh(core_axis_name="c", subcore_axis_name="s"),
    out_type=jax.ShapeDtypeStruct((R_TOT, D_E), jnp.float32),
    scratch_types=[
        pltpu.VMEM((C2,), jnp.int32),
        pltpu.VMEM((C2, D_E), jnp.float32),
        pltpu.SemaphoreType.DMA,
    ],
)
def _sc_gather(rows_hbm, tab_hbm, x_hbm, idx_v, buf_v, sem):
    cid = lax.axis_index("c")
    sid = lax.axis_index("s")
    wid = sid * NC + cid
    r0 = wid * RPW

    def chunk_body(k, carry):
        off = r0 + k * C2
        pltpu.sync_copy(rows_hbm.at[pl.ds(off, C2)], idx_v)
        pltpu.async_copy(tab_hbm.at[idx_v], buf_v, sem).wait()
        pltpu.sync_copy(buf_v, x_hbm.at[pl.ds(off, C2)])
        return carry

    lax.fori_loop(0, NCH2, chunk_body, 0)


def _tc_matmul(x_flat, w2, bias2):
    def body(x_ref, w_ref, b_ref, o_ref):
        xb = x_ref.reshape(TB, NUM_E * D_E)[...]
        o_ref[...] = jnp.dot(
            xb.astype(jnp.bfloat16), w_ref[...],
            preferred_element_type=jnp.float32) + b_ref[...]

    return pl.pallas_call(
        body,
        grid=(TOKENS // TB,),
        in_specs=[
            pl.BlockSpec((NUM_E * TB, D_E), lambda t: (t, 0)),
            pl.BlockSpec((NUM_E * D_E, D_O), lambda t: (0, 0)),
            pl.BlockSpec((1, D_O), lambda t: (0, 0)),
        ],
        out_specs=pl.BlockSpec((TB, D_O), lambda t: (t, 0)),
        out_shape=jax.ShapeDtypeStruct((TOKENS, D_O), jnp.float32),
    )(x_flat, w2, bias2)


def kernel(inputs, tables, W, b):
    tab2 = tables.reshape(NUM_E * VOC, D_E)
    idx2 = inputs.reshape(TOKENS, NUM_E)
    src = (idx2 + jnp.arange(NUM_E, dtype=jnp.int32) * VOC).reshape(-1)
    x = _sc_gather(src, tab2)
    out = _tc_matmul(x, W.astype(jnp.bfloat16), b.reshape(1, D_O))
    return out.reshape(B_SZ, L_SZ, D_O)


# speedup vs baseline: 6.4306x; 1.4183x over previous
import functools

import jax
import jax.numpy as jnp
from jax import lax
from jax.experimental import pallas as pl
from jax.experimental.pallas import tpu as pltpu
from jax.experimental.pallas import tpu_sc as plsc

NUM_E = 26
VOC = 1000
D_E = 128
D_O = 512
B_SZ = 4096
L_SZ = 20
TOKENS = B_SZ * L_SZ
NC = 2
NS = 16
NW = NC * NS
R_TOT = NUM_E * TOKENS
RPW = R_TOT // NW
C2 = 128
NCH2 = RPW // C2
NBUF = 4
IDXB = 32
NOUT = NCH2 // NBUF
TB = 512


@functools.partial(
    pl.kernel,
    mesh=plsc.VectorSubcoreMesh(core_axis_name="c", subcore_axis_name="s"),
    out_type=jax.ShapeDtypeStruct((R_TOT, D_E), jnp.float32),
    scratch_types=[
        pltpu.VMEM((IDXB, C2), jnp.int32),
        pltpu.VMEM((NBUF, C2, D_E), jnp.float32),
        [pltpu.SemaphoreType.DMA] * NBUF,
        [pltpu.SemaphoreType.DMA] * NBUF,
    ],
)
def _sc_gather(rows_hbm, tab_hbm, x_hbm, idx_v, bufs_v, gsems, wsems):
    cid = lax.axis_index("c")
    sid = lax.axis_index("s")
    wid = sid * NC + cid
    r0 = wid * RPW

    def outer_body(j, carry):
        @pl.when(j % (IDXB // NBUF) == 0)
        def _():
            base = pl.multiple_of((r0 // C2) + j * NBUF, 8)
            pltpu.sync_copy(rows_hbm.at[pl.ds(base, IDXB)], idx_v)

        row = (j % (IDXB // NBUF)) * NBUF
        gathers = []
        for b in range(NBUF):
            off = r0 + (j * NBUF + b) * C2

            @pl.when(j > 0)
            def _(b=b, off=off):
                pltpu.make_async_copy(
                    bufs_v.at[b], x_hbm.at[pl.ds(off, C2)], wsems[b]
                ).wait()

            gathers.append(
                pltpu.async_copy(
                    tab_hbm.at[idx_v.at[row + b]], bufs_v.at[b], gsems[b]
                )
            )
        for b in range(NBUF):
            off = r0 + (j * NBUF + b) * C2
            gathers[b].wait()
            pltpu.async_copy(bufs_v.at[b], x_hbm.at[pl.ds(off, C2)], wsems[b])
        return carry

    lax.fori_loop(0, NOUT, outer_body, 0)
    for b in range(NBUF):
        pltpu.make_async_copy(
            bufs_v.at[b], x_hbm.at[pl.ds(r0, C2)], wsems[b]
        ).wait()


def _tc_matmul(x_flat, w2, bias2):
    def body(x_ref, w_ref, b_ref, o_ref):
        xb = x_ref.reshape(TB, NUM_E * D_E)[...]
        o_ref[...] = jnp.dot(
            xb.astype(jnp.bfloat16), w_ref[...],
            preferred_element_type=jnp.float32) + b_ref[...]

    return pl.pallas_call(
        body,
        grid=(TOKENS // TB,),
        in_specs=[
            pl.BlockSpec((NUM_E * TB, D_E), lambda t: (t, 0)),
            pl.BlockSpec((NUM_E * D_E, D_O), lambda t: (0, 0)),
            pl.BlockSpec((1, D_O), lambda t: (0, 0)),
        ],
        out_specs=pl.BlockSpec((TB, D_O), lambda t: (t, 0)),
        out_shape=jax.ShapeDtypeStruct((TOKENS, D_O), jnp.float32),
    )(x_flat, w2, bias2)


def kernel(inputs, tables, W, b):
    tab2 = tables.reshape(NUM_E * VOC, D_E)
    idx2 = inputs.reshape(TOKENS, NUM_E)
    src = (idx2 + jnp.arange(NUM_E, dtype=jnp.int32) * VOC).reshape(
        R_TOT // C2, C2)
    x = _sc_gather(src, tab2)
    out = _tc_matmul(x, W.astype(jnp.bfloat16), b.reshape(1, D_O))
    return out.reshape(B_SZ, L_SZ, D_O)


# speedup vs baseline: 7.2061x; 1.1206x over previous
import functools

import jax
import jax.numpy as jnp
from jax import lax
from jax.experimental import pallas as pl
from jax.experimental.pallas import tpu as pltpu
from jax.experimental.pallas import tpu_sc as plsc

NUM_E = 26
VOC = 1000
D_E = 128
D_O = 512
B_SZ = 4096
L_SZ = 20
TOKENS = B_SZ * L_SZ
NC = 2
NS = 16
NW = NC * NS
R_TOT = NUM_E * TOKENS
RPW = R_TOT // NW
C2 = 128
NCH2 = RPW // C2
NBUF = 4
IDXB = 32
NOUT = NCH2 // NBUF
TB = 640


@functools.partial(
    pl.kernel,
    mesh=plsc.VectorSubcoreMesh(core_axis_name="c", subcore_axis_name="s"),
    out_type=jax.ShapeDtypeStruct((R_TOT, D_E), jnp.float32),
    scratch_types=[
        pltpu.VMEM((IDXB, C2), jnp.int32),
        pltpu.VMEM((NBUF, C2, D_E), jnp.float32),
        [pltpu.SemaphoreType.DMA] * NBUF,
        [pltpu.SemaphoreType.DMA] * NBUF,
    ],
)
def _sc_gather(rows_hbm, tab_hbm, x_hbm, idx_v, bufs_v, gsems, wsems):
    cid = lax.axis_index("c")
    sid = lax.axis_index("s")
    wid = sid * NC + cid
    r0 = wid * RPW

    def outer_body(j, carry):
        @pl.when(j % (IDXB // NBUF) == 0)
        def _():
            base = pl.multiple_of((r0 // C2) + j * NBUF, 8)
            pltpu.sync_copy(rows_hbm.at[pl.ds(base, IDXB)], idx_v)

        row = (j % (IDXB // NBUF)) * NBUF
        gathers = []
        for b in range(NBUF):
            off = r0 + (j * NBUF + b) * C2

            @pl.when(j > 0)
            def _(b=b, off=off):
                pltpu.make_async_copy(
                    bufs_v.at[b], x_hbm.at[pl.ds(off, C2)], wsems[b]
                ).wait()

            gathers.append(
                pltpu.async_copy(
                    tab_hbm.at[idx_v.at[row + b]], bufs_v.at[b], gsems[b]
                )
            )
        for b in range(NBUF):
            off = r0 + (j * NBUF + b) * C2
            gathers[b].wait()
            pltpu.async_copy(bufs_v.at[b], x_hbm.at[pl.ds(off, C2)], wsems[b])
        return carry

    lax.fori_loop(0, NOUT, outer_body, 0)
    for b in range(NBUF):
        pltpu.make_async_copy(
            bufs_v.at[b], x_hbm.at[pl.ds(r0, C2)], wsems[b]
        ).wait()


def _tc_matmul(x_flat, w2, bias2):
    def body(x_ref, w_ref, b_ref, o_ref):
        xb = x_ref.reshape(TB, NUM_E * D_E)[...]
        part = jnp.dot(
            xb.astype(jnp.bfloat16), w_ref[...],
            preferred_element_type=jnp.float32) + b_ref[...]
        for g in range(TB // L_SZ):
            o_ref[g] = part[g * L_SZ:(g + 1) * L_SZ]

    return pl.pallas_call(
        body,
        grid=(TOKENS // TB,),
        in_specs=[
            pl.BlockSpec((NUM_E * TB, D_E), lambda t: (t, 0)),
            pl.BlockSpec((NUM_E * D_E, D_O), lambda t: (0, 0)),
            pl.BlockSpec((1, D_O), lambda t: (0, 0)),
        ],
        out_specs=pl.BlockSpec((TB // L_SZ, L_SZ, D_O), lambda t: (t, 0, 0)),
        out_shape=jax.ShapeDtypeStruct((B_SZ, L_SZ, D_O), jnp.float32),
    )(x_flat, w2, bias2)


def kernel(inputs, tables, W, b):
    tab2 = tables.reshape(NUM_E * VOC, D_E)
    idx2 = inputs.reshape(TOKENS, NUM_E)
    src = (idx2 + jnp.arange(NUM_E, dtype=jnp.int32) * VOC).reshape(
        R_TOT // C2, C2)
    x = _sc_gather(src, tab2)
    return _tc_matmul(x, W.astype(jnp.bfloat16), b.reshape(1, D_O))
